# baseline (speedup 1.0000x reference)
import functools

import jax
import jax.numpy as jnp
from jax import lax
from jax.experimental import pallas as pl
from jax.experimental.pallas import tpu as pltpu
from jax.experimental.pallas import tpu_sc as plsc

_B = 16
_T = 1024
_TOKENS = _B * _T
_DIM = 64
_CODES = 1024

_NUM_WORKERS = 32
_ROWS_PER_WORKER = _TOKENS // _NUM_WORKERS
_SPLIT = _T // _ROWS_PER_WORKER


def _argmin_body(zt_ref, e_ref, rev_ref, ind_ref, acc_ref):
    zt = zt_ref[0]
    e = e_ref[...]
    zg2 = jnp.dot(e + e, zt, preferred_element_type=jnp.float32)
    rn = jnp.sum(zt * zt, axis=0, keepdims=True)
    en = jnp.sum(e * e, axis=1, keepdims=True)
    dist = (rn - zg2) + en
    m = jnp.min(dist, axis=0, keepdims=True)
    rev = rev_ref[...]
    picked = jnp.max(jnp.where(dist == m, rev, 0.0), axis=0)
    ind = _CODES - picked.astype(jnp.int32)
    ind_ref[...] = ind.reshape(1, 1, _T)

    @pl.when(pl.program_id(0) == 0)
    def _init():
        acc_ref[0, 0] = 0.0

    acc_ref[0, 0] += jnp.sum(m)


def _tc_argmin(zt, embed_weight, rev):
    return pl.pallas_call(
        _argmin_body,
        grid=(_B,),
        in_specs=[
            pl.BlockSpec((1, _DIM, _T), lambda i: (i, 0, 0)),
            pl.BlockSpec((_CODES, _DIM), lambda i: (0, 0)),
            pl.BlockSpec((_CODES, 1), lambda i: (0, 0)),
        ],
        out_specs=[
            pl.BlockSpec((1, 1, _T), lambda i: (i, 0, 0)),
            pl.BlockSpec((1, 1), lambda i: (0, 0), memory_space=pltpu.SMEM),
        ],
        out_shape=[
            jax.ShapeDtypeStruct((_B, 1, _T), jnp.int32),
            jax.ShapeDtypeStruct((1, 1), jnp.float32),
        ],
    )(zt, embed_weight, rev)


@functools.cache
def _build_sc_gather():
    @functools.partial(
        pl.kernel,
        out_type=jax.ShapeDtypeStruct((_B, _DIM, _T), jnp.float32),
        mesh=plsc.VectorSubcoreMesh(core_axis_name="c", subcore_axis_name="s"),
        scratch_types=[
            pltpu.VMEM((_ROWS_PER_WORKER,), jnp.int32),
            pltpu.VMEM((_ROWS_PER_WORKER, _DIM), jnp.float32),
            pltpu.VMEM((_DIM, _ROWS_PER_WORKER), jnp.float32),
            pltpu.SemaphoreType.DMA,
        ],
        compiler_params=pltpu.CompilerParams(
            use_tc_tiling_on_sc=False, needs_layout_passes=False
        ),
    )
    def _sc_gather(table_hbm, idx_hbm, out_hbm, idx_v, rows_v, rows_t, sem):
        wid = lax.axis_index("s") * 2 + lax.axis_index("c")
        base = wid * _ROWS_PER_WORKER
        pltpu.sync_copy(idx_hbm.at[pl.ds(base, _ROWS_PER_WORKER)], idx_v)
        pltpu.async_copy(table_hbm.at[idx_v], rows_v, sem).wait()
        lane = lax.iota(jnp.int32, 16)

        @pl.loop(0, _ROWS_PER_WORKER // 16)
        def _tile(g):
            row = g * 16 + lane
            for d in range(_DIM):
                col_d = jnp.full((16,), d, jnp.int32)
                v = plsc.load_gather(rows_v, [row, col_d])
                rows_t[d, pl.ds(g * 16, 16)] = v

        b = wid // _SPLIT
        col = (wid % _SPLIT) * _ROWS_PER_WORKER
        pltpu.sync_copy(rows_t, out_hbm.at[b, :, pl.ds(col, _ROWS_PER_WORKER)])

    return _sc_gather


def kernel(z_e, embed_weight):
    zt = lax.transpose(z_e, (0, 2, 1))
    rev = (_CODES - jnp.arange(_CODES, dtype=jnp.float32)).reshape(_CODES, 1)
    ind3, dist_sum = _tc_argmin(zt, embed_weight, rev)
    ind_flat = ind3.reshape(-1)
    zq_t = _build_sc_gather()(embed_weight, ind_flat)
    z_q = lax.transpose(zq_t, (0, 2, 1))
    diff = (1.25 / z_e.size) * dist_sum[0, 0]
    embed_ind = ind3.reshape(_B, _T)
    return (z_q, diff, embed_ind)


# speedup vs baseline: 1.2602x; 1.2602x over previous
import functools

import jax
import jax.numpy as jnp
from jax import lax
from jax.experimental import pallas as pl
from jax.experimental.pallas import tpu as pltpu
from jax.experimental.pallas import tpu_sc as plsc

_B = 16
_T = 1024
_TOKENS = _B * _T
_DIM = 64
_CODES = 1024

_NUM_WORKERS = 32
_ROWS_PER_WORKER = _TOKENS // _NUM_WORKERS
_SPLIT = _T // _ROWS_PER_WORKER


def _argmin_body(zt_ref, e_ref, rev_ref, ind_ref, acc_ref):
    zt = zt_ref[0]
    e = e_ref[...]
    zg2 = jnp.dot(e + e, zt, preferred_element_type=jnp.float32)
    rn = jnp.sum(zt * zt, axis=0, keepdims=True)
    en = jnp.sum(e * e, axis=1, keepdims=True)
    dist = (rn - zg2) + en
    m = jnp.min(dist, axis=0, keepdims=True)
    rev = rev_ref[...]
    picked = jnp.max(jnp.where(dist == m, rev, 0.0), axis=0)
    ind = _CODES - picked.astype(jnp.int32)
    ind_ref[...] = ind.reshape(1, 1, _T)

    @pl.when(pl.program_id(0) == 0)
    def _init():
        acc_ref[0, 0] = 0.0

    acc_ref[0, 0] += jnp.sum(m)


def _tc_argmin(zt, embed_weight, rev):
    return pl.pallas_call(
        _argmin_body,
        grid=(_B,),
        in_specs=[
            pl.BlockSpec((1, _DIM, _T), lambda i: (i, 0, 0)),
            pl.BlockSpec((_CODES, _DIM), lambda i: (0, 0)),
            pl.BlockSpec((_CODES, 1), lambda i: (0, 0)),
        ],
        out_specs=[
            pl.BlockSpec((1, 1, _T), lambda i: (i, 0, 0)),
            pl.BlockSpec((1, 1), lambda i: (0, 0), memory_space=pltpu.SMEM),
        ],
        out_shape=[
            jax.ShapeDtypeStruct((_B, 1, _T), jnp.int32),
            jax.ShapeDtypeStruct((1, 1), jnp.float32),
        ],
    )(zt, embed_weight, rev)


@functools.cache
def _build_sc_gather():
    @functools.partial(
        pl.kernel,
        out_type=jax.ShapeDtypeStruct((_B, _DIM, _T), jnp.float32),
        mesh=plsc.VectorSubcoreMesh(core_axis_name="c", subcore_axis_name="s"),
        scratch_types=[
            pltpu.VMEM((_ROWS_PER_WORKER,), jnp.int32),
            pltpu.VMEM((_ROWS_PER_WORKER, _DIM), jnp.float32),
            pltpu.VMEM((_DIM, _ROWS_PER_WORKER + 1), jnp.float32),
            pltpu.SemaphoreType.DMA,
        ],
        compiler_params=pltpu.CompilerParams(
            use_tc_tiling_on_sc=False, needs_layout_passes=False
        ),
    )
    def _sc_gather(table_hbm, idx_hbm, out_hbm, idx_v, rows_v, rows_t, sem):
        wid = lax.axis_index("s") * 2 + lax.axis_index("c")
        base = wid * _ROWS_PER_WORKER
        pltpu.sync_copy(idx_hbm.at[pl.ds(base, _ROWS_PER_WORKER)], idx_v)
        pltpu.async_copy(table_hbm.at[idx_v], rows_v, sem).wait()
        lane = lax.iota(jnp.int32, 16)
        d_idx = [lane + 16 * k for k in range(_DIM // 16)]

        @pl.loop(0, _ROWS_PER_WORKER)
        def _tok(t):
            t_col = jnp.zeros((16,), jnp.int32) + t
            for k in range(_DIM // 16):
                v = rows_v[t, pl.ds(16 * k, 16)]
                plsc.store_scatter(rows_t, [d_idx[k], t_col], v)

        b = wid // _SPLIT
        col = (wid % _SPLIT) * _ROWS_PER_WORKER
        pltpu.sync_copy(
            rows_t.at[:, pl.ds(0, _ROWS_PER_WORKER)],
            out_hbm.at[b, :, pl.ds(col, _ROWS_PER_WORKER)],
        )

    return _sc_gather


def kernel(z_e, embed_weight):
    zt = lax.transpose(z_e, (0, 2, 1))
    rev = (_CODES - jnp.arange(_CODES, dtype=jnp.float32)).reshape(_CODES, 1)
    ind3, dist_sum = _tc_argmin(zt, embed_weight, rev)
    ind_flat = ind3.reshape(-1)
    zq_t = _build_sc_gather()(embed_weight, ind_flat)
    z_q = lax.transpose(zq_t, (0, 2, 1))
    diff = (1.25 / z_e.size) * dist_sum[0, 0]
    embed_ind = ind3.reshape(_B, _T)
    return (z_q, diff, embed_ind)


# speedup vs baseline: 1.3311x; 1.0562x over previous
import functools

import jax
import jax.numpy as jnp
from jax import lax
from jax.experimental import pallas as pl
from jax.experimental.pallas import tpu as pltpu
from jax.experimental.pallas import tpu_sc as plsc

_B = 16
_T = 1024
_TOKENS = _B * _T
_DIM = 64
_CODES = 1024

_BATCH_PER_BLOCK = 2
_NUM_BLOCKS = _B // _BATCH_PER_BLOCK

_NUM_WORKERS = 32
_ROWS_PER_WORKER = _TOKENS // _NUM_WORKERS
_SPLIT = _T // _ROWS_PER_WORKER


def _argmin_body(zt_ref, et_ref, ind_ref, acc_ref):
    et = et_ref[...]
    en = jnp.sum(et * et, axis=0, keepdims=True).T
    et2 = et + et
    rev = jnp.float32(_CODES) - lax.broadcasted_iota(
        jnp.int32, (_CODES, 1), 0
    ).astype(jnp.float32)

    @pl.when(pl.program_id(0) == 0)
    def _init():
        acc_ref[0, 0] = 0.0

    for r in range(_BATCH_PER_BLOCK):
        zt = zt_ref[r]
        zg2 = lax.dot_general(
            et2, zt, (((0,), (0,)), ((), ())),
            preferred_element_type=jnp.float32,
        )
        rn = jnp.sum(zt * zt, axis=0, keepdims=True)
        dist = (rn - zg2) + en
        m = jnp.min(dist, axis=0, keepdims=True)
        picked = jnp.max(jnp.where(dist == m, rev, 0.0), axis=0)
        ind = _CODES - picked.astype(jnp.int32)
        ind_ref[r] = ind.reshape(1, _T)
        acc_ref[0, 0] += jnp.sum(m)


def _tc_argmin(zt, et):
    return pl.pallas_call(
        _argmin_body,
        grid=(_NUM_BLOCKS,),
        in_specs=[
            pl.BlockSpec((_BATCH_PER_BLOCK, _DIM, _T), lambda i: (i, 0, 0)),
            pl.BlockSpec((_DIM, _CODES), lambda i: (0, 0)),
        ],
        out_specs=[
            pl.BlockSpec((_BATCH_PER_BLOCK, 1, _T), lambda i: (i, 0, 0)),
            pl.BlockSpec((1, 1), lambda i: (0, 0), memory_space=pltpu.SMEM),
        ],
        out_shape=[
            jax.ShapeDtypeStruct((_B, 1, _T), jnp.int32),
            jax.ShapeDtypeStruct((1, 1), jnp.float32),
        ],
    )(zt, et)


@functools.cache
def _build_sc_gather():
    @functools.partial(
        pl.kernel,
        out_type=jax.ShapeDtypeStruct((_B, _DIM, _T), jnp.float32),
        mesh=plsc.VectorSubcoreMesh(core_axis_name="c", subcore_axis_name="s"),
        scratch_types=[
            pltpu.VMEM((_ROWS_PER_WORKER,), jnp.int32),
            pltpu.VMEM((_ROWS_PER_WORKER, _DIM), jnp.float32),
            pltpu.VMEM((_DIM, _ROWS_PER_WORKER + 1), jnp.float32),
            pltpu.SemaphoreType.DMA,
        ],
        compiler_params=pltpu.CompilerParams(
            use_tc_tiling_on_sc=False, needs_layout_passes=False
        ),
    )
    def _sc_gather(table_hbm, idx_hbm, out_hbm, idx_v, rows_v, rows_t, sem):
        wid = lax.axis_index("s") * 2 + lax.axis_index("c")
        base = wid * _ROWS_PER_WORKER
        pltpu.sync_copy(idx_hbm.at[pl.ds(base, _ROWS_PER_WORKER)], idx_v)
        pltpu.async_copy(table_hbm.at[idx_v], rows_v, sem).wait()
        lane = lax.iota(jnp.int32, 16)
        d_idx = [lane + 16 * k for k in range(_DIM // 16)]

        @pl.loop(0, _ROWS_PER_WORKER // 4)
        def _tok(t4):
            t0 = t4 * 4
            for u in range(4):
                t_col = jnp.zeros((16,), jnp.int32) + (t0 + u)
                for k in range(_DIM // 16):
                    v = rows_v[t0 + u, pl.ds(16 * k, 16)]
                    plsc.store_scatter(rows_t, [d_idx[k], t_col], v)

        b = wid // _SPLIT
        col = (wid % _SPLIT) * _ROWS_PER_WORKER
        pltpu.sync_copy(
            rows_t.at[:, pl.ds(0, _ROWS_PER_WORKER)],
            out_hbm.at[b, :, pl.ds(col, _ROWS_PER_WORKER)],
        )

    return _sc_gather


def kernel(z_e, embed_weight):
    zt = lax.transpose(z_e, (0, 2, 1))
    et = lax.transpose(embed_weight, (1, 0))
    ind3, dist_sum = _tc_argmin(zt, et)
    ind_flat = ind3.reshape(-1)
    zq_t = _build_sc_gather()(embed_weight, ind_flat)
    z_q = lax.transpose(zq_t, (0, 2, 1))
    diff = (1.25 / z_e.size) * dist_sum[0, 0]
    embed_ind = ind3.reshape(_B, _T)
    return (z_q, diff, embed_ind)


# speedup vs baseline: 1.3561x; 1.0188x over previous
import functools

import jax
import jax.numpy as jnp
from jax import lax
from jax.experimental import pallas as pl
from jax.experimental.pallas import tpu as pltpu
from jax.experimental.pallas import tpu_sc as plsc

_B = 16
_T = 1024
_TOKENS = _B * _T
_DIM = 64
_CODES = 1024

_BATCH_PER_BLOCK = 4
_NUM_BLOCKS = _B // _BATCH_PER_BLOCK

_NUM_WORKERS = 32
_ROWS_PER_WORKER = _TOKENS // _NUM_WORKERS
_SPLIT = _T // _ROWS_PER_WORKER


def _argmin_body(zt_ref, et_ref, ind_ref, acc_ref):
    et = et_ref[...]
    en = jnp.sum(et * et, axis=0, keepdims=True).T
    et2 = et + et
    rev = jnp.float32(_CODES) - lax.broadcasted_iota(
        jnp.int32, (_CODES, 1), 0
    ).astype(jnp.float32)

    @pl.when(pl.program_id(0) == 0)
    def _init():
        acc_ref[0, 0] = 0.0

    for r in range(_BATCH_PER_BLOCK):
        zt = zt_ref[r]
        zg2 = lax.dot_general(
            et2, zt, (((0,), (0,)), ((), ())),
            preferred_element_type=jnp.float32,
        )
        rn = jnp.sum(zt * zt, axis=0, keepdims=True)
        dist = (rn - zg2) + en
        m = jnp.min(dist, axis=0, keepdims=True)
        picked = jnp.max(jnp.where(dist == m, rev, 0.0), axis=0)
        ind = _CODES - picked.astype(jnp.int32)
        ind_ref[r] = ind.reshape(1, _T)
        acc_ref[0, 0] += jnp.sum(m)


def _tc_argmin(zt, et):
    return pl.pallas_call(
        _argmin_body,
        grid=(_NUM_BLOCKS,),
        in_specs=[
            pl.BlockSpec((_BATCH_PER_BLOCK, _DIM, _T), lambda i: (i, 0, 0)),
            pl.BlockSpec((_DIM, _CODES), lambda i: (0, 0)),
        ],
        out_specs=[
            pl.BlockSpec((_BATCH_PER_BLOCK, 1, _T), lambda i: (i, 0, 0)),
            pl.BlockSpec((1, 1), lambda i: (0, 0), memory_space=pltpu.SMEM),
        ],
        out_shape=[
            jax.ShapeDtypeStruct((_B, 1, _T), jnp.int32),
            jax.ShapeDtypeStruct((1, 1), jnp.float32),
        ],
    )(zt, et)


@functools.cache
def _build_sc_gather():
    @functools.partial(
        pl.kernel,
        out_type=jax.ShapeDtypeStruct((_B, _DIM, _T), jnp.float32),
        mesh=plsc.VectorSubcoreMesh(core_axis_name="c", subcore_axis_name="s"),
        scratch_types=[
            pltpu.VMEM((_ROWS_PER_WORKER,), jnp.int32),
            pltpu.VMEM((_ROWS_PER_WORKER, _DIM), jnp.float32),
            pltpu.VMEM((_DIM, _ROWS_PER_WORKER + 8), jnp.float32),
            pltpu.SemaphoreType.DMA,
        ],
        compiler_params=pltpu.CompilerParams(
            use_tc_tiling_on_sc=False, needs_layout_passes=False
        ),
    )
    def _sc_gather(table_hbm, idx_hbm, out_hbm, idx_v, rows_v, rows_t, sem):
        wid = lax.axis_index("s") * 2 + lax.axis_index("c")
        base = wid * _ROWS_PER_WORKER
        pltpu.sync_copy(idx_hbm.at[pl.ds(base, _ROWS_PER_WORKER)], idx_v)
        pltpu.async_copy(table_hbm.at[idx_v], rows_v, sem).wait()
        lane = lax.iota(jnp.int32, 16)
        d_idx = [lane + 16 * k for k in range(_DIM // 16)]

        @pl.loop(0, _ROWS_PER_WORKER // 4)
        def _tok(t4):
            t0 = t4 * 4
            for u in range(4):
                t_col = jnp.zeros((16,), jnp.int32) + (t0 + u)
                for k in range(_DIM // 16):
                    v = rows_v[t0 + u, pl.ds(16 * k, 16)]
                    plsc.store_scatter(rows_t, [d_idx[k], t_col], v)

        b = wid // _SPLIT
        col = (wid % _SPLIT) * _ROWS_PER_WORKER
        pltpu.sync_copy(
            rows_t.at[:, pl.ds(0, _ROWS_PER_WORKER)],
            out_hbm.at[b, :, pl.ds(col, _ROWS_PER_WORKER)],
        )

    return _sc_gather


def kernel(z_e, embed_weight):
    zt = lax.transpose(z_e, (0, 2, 1))
    et = lax.transpose(embed_weight, (1, 0))
    ind3, dist_sum = _tc_argmin(zt, et)
    ind_flat = ind3.reshape(-1)
    zq_t = _build_sc_gather()(embed_weight, ind_flat)
    z_q = lax.transpose(zq_t, (0, 2, 1))
    diff = (1.25 / z_e.size) * dist_sum[0, 0]
    embed_ind = ind3.reshape(_B, _T)
    return (z_q, diff, embed_ind)


# speedup vs baseline: 1.4983x; 1.1048x over previous
import functools

import jax
import jax.numpy as jnp
from jax import lax
from jax.experimental import pallas as pl
from jax.experimental.pallas import tpu as pltpu
from jax.experimental.pallas import tpu_sc as plsc

_B = 16
_T = 1024
_TOKENS = _B * _T
_DIM = 64
_CODES = 1024

_BATCH_PER_BLOCK = 4
_NUM_BLOCKS = _B // _BATCH_PER_BLOCK

_NUM_WORKERS = 32
_ROWS_PER_WORKER = _TOKENS // _NUM_WORKERS
_SPLIT = _T // _ROWS_PER_WORKER


def _argmin_body(zt_ref, et_ref, ind_ref, acc_ref):
    et = et_ref[...]
    en = jnp.sum(et * et, axis=0, keepdims=True).T
    et2 = et + et
    rev = jnp.float32(_CODES) - lax.broadcasted_iota(
        jnp.int32, (_CODES, 1), 0
    ).astype(jnp.float32)

    @pl.when(pl.program_id(0) == 0)
    def _init():
        acc_ref[0, 0] = 0.0

    for r in range(_BATCH_PER_BLOCK):
        zt = zt_ref[r]
        zg2 = lax.dot_general(
            et2, zt, (((0,), (0,)), ((), ())),
            preferred_element_type=jnp.float32,
        )
        rn = jnp.sum(zt * zt, axis=0, keepdims=True)
        dist = (rn - zg2) + en
        m = jnp.min(dist, axis=0, keepdims=True)
        picked = jnp.max(jnp.where(dist == m, rev, 0.0), axis=0)
        ind = _CODES - picked.astype(jnp.int32)
        ind_ref[r] = ind.reshape(1, _T)
        acc_ref[0, 0] += jnp.sum(m)


def _tc_argmin(zt, et):
    return pl.pallas_call(
        _argmin_body,
        grid=(_NUM_BLOCKS,),
        in_specs=[
            pl.BlockSpec((_BATCH_PER_BLOCK, _DIM, _T), lambda i: (i, 0, 0)),
            pl.BlockSpec((_DIM, _CODES), lambda i: (0, 0)),
        ],
        out_specs=[
            pl.BlockSpec((_BATCH_PER_BLOCK, 1, _T), lambda i: (i, 0, 0)),
            pl.BlockSpec((1, 1), lambda i: (0, 0), memory_space=pltpu.SMEM),
        ],
        out_shape=[
            jax.ShapeDtypeStruct((_B, 1, _T), jnp.int32),
            jax.ShapeDtypeStruct((1, 1), jnp.float32),
        ],
    )(zt, et)


@functools.cache
def _build_sc_gather():
    @functools.partial(
        pl.kernel,
        out_type=jax.ShapeDtypeStruct((_B, _DIM, _T), jnp.float32),
        mesh=plsc.VectorSubcoreMesh(core_axis_name="c", subcore_axis_name="s"),
        scratch_types=[
            pltpu.VMEM((_ROWS_PER_WORKER,), jnp.int32),
            pltpu.VMEM((_ROWS_PER_WORKER, _DIM), jnp.float32),
            pltpu.VMEM((_DIM, _ROWS_PER_WORKER + 8), jnp.float32),
            pltpu.SemaphoreType.DMA,
        ],
        compiler_params=pltpu.CompilerParams(
            use_tc_tiling_on_sc=False, needs_layout_passes=False
        ),
    )
    def _sc_gather(table_hbm, idx_hbm, out_hbm, idx_v, rows_v, rows_t, sem):
        wid = lax.axis_index("s") * 2 + lax.axis_index("c")
        base = wid * _ROWS_PER_WORKER
        pltpu.sync_copy(idx_hbm.at[pl.ds(base, _ROWS_PER_WORKER)], idx_v)
        pltpu.async_copy(table_hbm.at[idx_v], rows_v, sem).wait()
        lane = lax.iota(jnp.int32, 16)
        d_idx = [lane + 16 * k for k in range(_DIM // 16)]

        @plsc.parallel_loop(0, _ROWS_PER_WORKER, unroll=4)
        def _tok(t):
            t_col = jnp.zeros((16,), jnp.int32) + t
            for k in range(_DIM // 16):
                v = rows_v[t, pl.ds(16 * k, 16)]
                plsc.store_scatter(rows_t, [d_idx[k], t_col], v)

        b = wid // _SPLIT
        col = (wid % _SPLIT) * _ROWS_PER_WORKER
        pltpu.sync_copy(
            rows_t.at[:, pl.ds(0, _ROWS_PER_WORKER)],
            out_hbm.at[b, :, pl.ds(col, _ROWS_PER_WORKER)],
        )

    return _sc_gather


def kernel(z_e, embed_weight):
    zt = lax.transpose(z_e, (0, 2, 1))
    et = lax.transpose(embed_weight, (1, 0))
    ind3, dist_sum = _tc_argmin(zt, et)
    ind_flat = ind3.reshape(-1)
    zq_t = _build_sc_gather()(embed_weight, ind_flat)
    z_q = lax.transpose(zq_t, (0, 2, 1))
    diff = (1.25 / z_e.size) * dist_sum[0, 0]
    embed_ind = ind3.reshape(_B, _T)
    return (z_q, diff, embed_ind)


# speedup vs baseline: 1.5072x; 1.0060x over previous
import functools

import jax
import jax.numpy as jnp
from jax import lax
from jax.experimental import pallas as pl
from jax.experimental.pallas import tpu as pltpu
from jax.experimental.pallas import tpu_sc as plsc

_B = 16
_T = 1024
_TOKENS = _B * _T
_DIM = 64
_CODES = 1024

_BATCH_PER_BLOCK = 8
_NUM_BLOCKS = _B // _BATCH_PER_BLOCK

_NUM_WORKERS = 32
_ROWS_PER_WORKER = _TOKENS // _NUM_WORKERS
_SPLIT = _T // _ROWS_PER_WORKER


def _argmin_body(zt_ref, et_ref, ind_ref, acc_ref):
    et = et_ref[...]
    en = jnp.sum(et * et, axis=0, keepdims=True).T
    et2 = et + et
    rev = jnp.float32(_CODES) - lax.broadcasted_iota(
        jnp.int32, (_CODES, 1), 0
    ).astype(jnp.float32)

    @pl.when(pl.program_id(0) == 0)
    def _init():
        acc_ref[0, 0] = 0.0

    for r in range(_BATCH_PER_BLOCK):
        zt = zt_ref[r]
        zg2 = lax.dot_general(
            et2, zt, (((0,), (0,)), ((), ())),
            preferred_element_type=jnp.float32,
        )
        rn = jnp.sum(zt * zt, axis=0, keepdims=True)
        dist = (rn - zg2) + en
        m = jnp.min(dist, axis=0, keepdims=True)
        picked = jnp.max(jnp.where(dist == m, rev, 0.0), axis=0)
        ind = _CODES - picked.astype(jnp.int32)
        ind_ref[r] = ind.reshape(1, _T)
        acc_ref[0, 0] += jnp.sum(m)


def _tc_argmin(zt, et):
    return pl.pallas_call(
        _argmin_body,
        grid=(_NUM_BLOCKS,),
        in_specs=[
            pl.BlockSpec((_BATCH_PER_BLOCK, _DIM, _T), lambda i: (i, 0, 0)),
            pl.BlockSpec((_DIM, _CODES), lambda i: (0, 0)),
        ],
        out_specs=[
            pl.BlockSpec((_BATCH_PER_BLOCK, 1, _T), lambda i: (i, 0, 0)),
            pl.BlockSpec((1, 1), lambda i: (0, 0), memory_space=pltpu.SMEM),
        ],
        out_shape=[
            jax.ShapeDtypeStruct((_B, 1, _T), jnp.int32),
            jax.ShapeDtypeStruct((1, 1), jnp.float32),
        ],
    )(zt, et)


@functools.cache
def _build_sc_gather():
    @functools.partial(
        pl.kernel,
        out_type=jax.ShapeDtypeStruct((_B, _DIM, _T), jnp.float32),
        mesh=plsc.VectorSubcoreMesh(core_axis_name="c", subcore_axis_name="s"),
        scratch_types=[
            pltpu.VMEM((_ROWS_PER_WORKER,), jnp.int32),
            pltpu.VMEM((_ROWS_PER_WORKER, _DIM), jnp.float32),
            pltpu.VMEM((_DIM, _ROWS_PER_WORKER + 8), jnp.float32),
            pltpu.SemaphoreType.DMA,
        ],
        compiler_params=pltpu.CompilerParams(
            use_tc_tiling_on_sc=False, needs_layout_passes=False
        ),
    )
    def _sc_gather(table_hbm, idx_hbm, out_hbm, idx_v, rows_v, rows_t, sem):
        wid = lax.axis_index("s") * 2 + lax.axis_index("c")
        base = wid * _ROWS_PER_WORKER
        pltpu.sync_copy(idx_hbm.at[pl.ds(base, _ROWS_PER_WORKER)], idx_v)
        pltpu.async_copy(table_hbm.at[idx_v], rows_v, sem).wait()
        lane = lax.iota(jnp.int32, 16)
        d_idx = [lane + 16 * k for k in range(_DIM // 16)]

        @plsc.parallel_loop(0, _ROWS_PER_WORKER, unroll=4)
        def _tok(t):
            t_col = jnp.zeros((16,), jnp.int32) + t
            for k in range(_DIM // 16):
                v = rows_v[t, pl.ds(16 * k, 16)]
                plsc.store_scatter(rows_t, [d_idx[k], t_col], v)

        b = wid // _SPLIT
        col = (wid % _SPLIT) * _ROWS_PER_WORKER
        pltpu.sync_copy(
            rows_t.at[:, pl.ds(0, _ROWS_PER_WORKER)],
            out_hbm.at[b, :, pl.ds(col, _ROWS_PER_WORKER)],
        )

    return _sc_gather


def kernel(z_e, embed_weight):
    zt = lax.transpose(z_e, (0, 2, 1))
    et = lax.transpose(embed_weight, (1, 0))
    ind3, dist_sum = _tc_argmin(zt, et)
    ind_flat = ind3.reshape(-1)
    zq_t = _build_sc_gather()(embed_weight, ind_flat)
    z_q = lax.transpose(zq_t, (0, 2, 1))
    diff = (1.25 / z_e.size) * dist_sum[0, 0]
    embed_ind = ind3.reshape(_B, _T)
    return (z_q, diff, embed_ind)
